# Initial kernel scaffold; baseline (speedup 1.0000x reference)
import jax
import jax.numpy as jnp
from jax import lax
from jax.experimental import pallas as pl
from jax.experimental.pallas import tpu as pltpu, tpu_sc as plsc

NU = 25000
HP = 25600
NN = 2 * HP
D = 64
E = 800000
NC, NS = 2, 16
CH = 1024
NCHUNK = 49
EPT = NCHUNK * CH
EPAD = NS * EPT
DUMP = NU
RPT = HP // NS
B = 4096
BPT = B // (NC * NS)

_mesh = plsc.VectorSubcoreMesh(core_axis_name="c", subcore_axis_name="s",
                               num_cores=NC, num_subcores=NS)


def _layer_body(src_hbm, dst_hbm, w_hbm, emb_hbm, out_hbm,
                srcb, dstb, wb, rows, sem, acc):
    c = lax.axis_index("c")
    s = lax.axis_index("s")

    @pl.loop(0, 64)
    def _zero_rows(k):
        for j in range(4):
            rows[0, k, pl.ds(j * 16, 16)] = jnp.zeros((16,), jnp.float32)

    abase = s * RPT

    @pl.loop(0, RPT // 64)
    def _zero_acc(i):
        pltpu.sync_copy(rows.at[0, pl.ds(0, 64)],
                        acc.at[pl.ds(abase + i * 64, 64)])

    plsc.subcore_barrier()

    dst_lo = c * NU

    @pl.loop(0, NCHUNK)
    def _chunk(ci):
        ebase = s * EPT + ci * CH
        for r in range(8):
            pltpu.sync_copy(src_hbm.at[pl.ds(ebase + r * 128, 128)],
                            srcb.at[r])
            pltpu.sync_copy(dst_hbm.at[pl.ds(ebase + r * 128, 128)],
                            dstb.at[r])
        pltpu.sync_copy(w_hbm.at[pl.ds(ebase, CH)], wb)

        for r in range(8):
            @pl.loop(0, 8)
            def _fix(g, r=r):
                sl = pl.ds(g * 16, 16)
                sv = srcb[r, sl]
                srcb[r, sl] = jnp.where(sv < NU, sv, sv + (HP - NU))
                dv = dstb[r, sl] - dst_lo
                ok = (dv >= 0) & (dv < NU)
                dstb[r, sl] = jnp.where(ok, dv, DUMP)

        gets = [pltpu.async_copy(emb_hbm.at[srcb.at[r]], rows.at[r], sem)
                for r in range(8)]
        for g in gets:
            g.wait()

        for r in range(8):
            @pl.loop(0, 128)
            def _scale(k, r=r):
                wv = jnp.full((16,), wb[r * 128 + k], jnp.float32)
                for j in range(4):
                    sl = pl.ds(j * 16, 16)
                    rows[r, k, sl] = rows[r, k, sl] * wv

        puts = [pltpu.async_copy(rows.at[r], acc.at[dstb.at[r]], sem,
                                 add=True)
                for r in range(8)]
        for p in puts:
            p.wait()

    plsc.subcore_barrier()

    pltpu.sync_copy(acc.at[pl.ds(abase, RPT)],
                    out_hbm.at[pl.ds(c * HP + abase, RPT)])


_layer = pl.kernel(
    _layer_body,
    out_type=jax.ShapeDtypeStruct((NN, D), jnp.float32),
    mesh=_mesh,
    scratch_types=[
        pltpu.VMEM((8, 128), jnp.int32),
        pltpu.VMEM((8, 128), jnp.int32),
        pltpu.VMEM((CH,), jnp.float32),
        pltpu.VMEM((8, 128, D), jnp.float32),
        pltpu.SemaphoreType.DMA,
        pltpu.VMEM_SHARED((HP, D), jnp.float32),
    ],
)


def _readout_body(users_hbm, items_hbm, ub_hbm, ib_hbm,
                  e0, e1, e2, e3, gamma_hbm,
                  ubuf, ibuf, irow, ubtab, ibtab,
                  sumU, sumI, tmp, bia, outb, sem):
    c = lax.axis_index("c")
    s = lax.axis_index("s")
    wid = s * NC + c
    bbase = wid * BPT

    pltpu.sync_copy(users_hbm.at[pl.ds(bbase, BPT)], ubuf)
    pltpu.sync_copy(items_hbm.at[pl.ds(bbase, BPT)], ibuf)
    pltpu.sync_copy(ub_hbm, ubtab)
    pltpu.sync_copy(ib_hbm, ibtab)

    @pl.loop(0, BPT // 16)
    def _mkrow(g):
        sl = pl.ds(g * 16, 16)
        irow[sl] = ibuf[sl] + HP

    hops = [e0, e1, e2, e3]
    pltpu.async_copy(hops[0].at[ubuf], sumU, sem).wait()
    pltpu.async_copy(hops[0].at[irow], sumI, sem).wait()
    for h in range(1, 4):
        pltpu.async_copy(hops[h].at[ubuf], tmp, sem).wait()

        @pl.loop(0, BPT)
        def _accU(b):
            for j in range(4):
                sl = pl.ds(j * 16, 16)
                sumU[b, sl] = sumU[b, sl] + tmp[b, sl]

        pltpu.async_copy(hops[h].at[irow], tmp, sem).wait()

        @pl.loop(0, BPT)
        def _accI(b):
            for j in range(4):
                sl = pl.ds(j * 16, 16)
                sumI[b, sl] = sumI[b, sl] + tmp[b, sl]

    @pl.loop(0, BPT // 16)
    def _bias(g):
        sl = pl.ds(g * 16, 16)
        uv = plsc.load_gather(ubtab, [ubuf[sl]])
        iv = plsc.load_gather(ibtab, [ibuf[sl]])
        bia[sl] = uv + iv

    @pl.loop(0, BPT)
    def _dot(b):
        accv = jnp.zeros((16,), jnp.float32)
        for j in range(4):
            sl = pl.ds(j * 16, 16)
            accv = accv + sumU[b, sl] * sumI[b, sl]
        outb[b] = jnp.sum(accv) * jnp.float32(1.0 / 16.0) + bia[b]

    pltpu.sync_copy(outb, gamma_hbm.at[pl.ds(bbase, BPT)])


_readout = pl.kernel(
    _readout_body,
    out_type=jax.ShapeDtypeStruct((B,), jnp.float32),
    mesh=_mesh,
    scratch_types=[
        pltpu.VMEM((BPT,), jnp.int32),
        pltpu.VMEM((BPT,), jnp.int32),
        pltpu.VMEM((BPT,), jnp.int32),
        pltpu.VMEM((NU,), jnp.float32),
        pltpu.VMEM((NU,), jnp.float32),
        pltpu.VMEM((BPT, D), jnp.float32),
        pltpu.VMEM((BPT, D), jnp.float32),
        pltpu.VMEM((BPT, D), jnp.float32),
        pltpu.VMEM((BPT,), jnp.float32),
        pltpu.VMEM((BPT,), jnp.float32),
        pltpu.SemaphoreType.DMA,
    ],
)


@jax.jit
def kernel(users, items, edge_index, graph_values,
           user_emb, item_emb, user_bias, item_bias):
    src = edge_index[0].astype(jnp.int32)
    dst = edge_index[1].astype(jnp.int32)
    pad = EPAD - E
    src_p = jnp.concatenate([src, jnp.zeros((pad,), jnp.int32)])
    dst_p = jnp.concatenate([dst, jnp.full((pad,), 2 * NU, jnp.int32)])
    w_p = jnp.concatenate([graph_values.astype(jnp.float32),
                           jnp.zeros((pad,), jnp.float32)])

    e0 = jnp.zeros((NN, D), jnp.float32)
    e0 = e0.at[:NU].set(user_emb).at[HP:HP + NU].set(item_emb)

    e1 = _layer(src_p, dst_p, w_p, e0)
    e2 = _layer(src_p, dst_p, w_p, e1)
    e3 = _layer(src_p, dst_p, w_p, e2)

    gamma = _readout(users.astype(jnp.int32), items.astype(jnp.int32),
                     user_bias[:, 0].astype(jnp.float32),
                     item_bias[:, 0].astype(jnp.float32),
                     e0, e1, e2, e3)
    return gamma


# speedup vs baseline: 2.8800x; 2.8800x over previous
import jax
import jax.numpy as jnp
from jax import lax
from jax.experimental import pallas as pl
from jax.experimental.pallas import tpu as pltpu, tpu_sc as plsc

NU = 25000
HP = 25600
NN = 2 * HP
D = 64
E = 800000
NC, NS = 2, 16
CH = 1024
NCHUNK = 49
EPT = NCHUNK * CH
EPAD = NS * EPT
DUMP = NU
RPT = HP // NS
B = 4096
BPT = B // (NC * NS)

_mesh = plsc.VectorSubcoreMesh(core_axis_name="c", subcore_axis_name="s",
                               num_cores=NC, num_subcores=NS)


def _layer_body(src_hbm, dst_hbm, w_hbm, emb_hbm, out_hbm,
                srcb, dstb, wb, rows, sem, acc):
    c = lax.axis_index("c")
    s = lax.axis_index("s")

    @pl.loop(0, 64)
    def _zero_rows(k):
        for j in range(4):
            rows[0, k, pl.ds(j * 16, 16)] = jnp.zeros((16,), jnp.float32)

    abase = s * RPT

    @pl.loop(0, RPT // 64)
    def _zero_acc(i):
        pltpu.sync_copy(rows.at[0, pl.ds(0, 64)],
                        acc.at[pl.ds(abase + i * 64, 64)])

    plsc.subcore_barrier()

    dst_lo = c * NU

    @pl.loop(0, NCHUNK)
    def _chunk(ci):
        ebase = s * EPT + ci * CH
        for r in range(8):
            pltpu.sync_copy(src_hbm.at[pl.ds(ebase + r * 128, 128)],
                            srcb.at[r])
            pltpu.sync_copy(dst_hbm.at[pl.ds(ebase + r * 128, 128)],
                            dstb.at[r])
        pltpu.sync_copy(w_hbm.at[pl.ds(ebase, CH)], wb.at[pl.ds(0, CH)])

        for r in range(8):
            @pl.loop(0, 8)
            def _fix(g, r=r):
                sl = pl.ds(g * 16, 16)
                sv = srcb[r, sl]
                srcb[r, sl] = jnp.where(sv < NU, sv, sv + (HP - NU))
                dv = dstb[r, sl] - dst_lo
                ok = (dv >= 0) & (dv < NU)
                dstb[r, sl] = jnp.where(ok, dv, DUMP)

        for r in range(8):
            b = r & 1
            pltpu.async_copy(emb_hbm.at[srcb.at[r]], rows.at[b], sem).wait()

            @pl.loop(0, 128)
            def _scale(k, r=r, b=b):
                wv = jnp.full((16,), wb[pl.ds(r * 128 + k, 16)][0],
                              jnp.float32)
                for j in range(4):
                    sl = pl.ds(j * 16, 16)
                    rows[b, k, sl] = rows[b, k, sl] * wv

            pltpu.async_copy(rows.at[b], acc.at[dstb.at[r]], sem,
                             add=True).wait()

    plsc.subcore_barrier()

    pltpu.sync_copy(acc.at[pl.ds(abase, RPT)],
                    out_hbm.at[pl.ds(c * HP + abase, RPT)])


_layer = pl.kernel(
    _layer_body,
    out_type=jax.ShapeDtypeStruct((NN, D), jnp.float32),
    mesh=_mesh,
    compiler_params=pltpu.CompilerParams(use_tc_tiling_on_sc=False,
                                        needs_layout_passes=False),
    scratch_types=[
        pltpu.VMEM((8, 128), jnp.int32),
        pltpu.VMEM((8, 128), jnp.int32),
        pltpu.VMEM((CH + 16,), jnp.float32),
        pltpu.VMEM((2, 128, D), jnp.float32),
        pltpu.SemaphoreType.DMA,
        pltpu.VMEM_SHARED((HP, D), jnp.float32),
    ],
)


def _readout_body(users_hbm, items_hbm, ub_hbm, ib_hbm,
                  e0, e1, e2, e3, gamma_hbm,
                  ubuf, ibuf, irow, ubtab, ibtab,
                  sumU, sumI, tmp, outb, sem):
    c = lax.axis_index("c")
    s = lax.axis_index("s")
    wid = s * NC + c
    bbase = wid * BPT

    pltpu.sync_copy(users_hbm.at[pl.ds(bbase, BPT)], ubuf)
    pltpu.sync_copy(items_hbm.at[pl.ds(bbase, BPT)], ibuf)
    pltpu.sync_copy(ub_hbm, ubtab.at[pl.ds(0, NU)])
    pltpu.sync_copy(ib_hbm, ibtab.at[pl.ds(0, NU)])

    @pl.loop(0, BPT // 16)
    def _mkrow(g):
        sl = pl.ds(g * 16, 16)
        irow[sl] = ibuf[sl] + HP

    hops = [e0, e1, e2, e3]
    pltpu.async_copy(hops[0].at[ubuf], sumU, sem).wait()
    pltpu.async_copy(hops[0].at[irow], sumI, sem).wait()
    for h in range(1, 4):
        pltpu.async_copy(hops[h].at[ubuf], tmp, sem).wait()

        @pl.loop(0, BPT)
        def _accU(b):
            for j in range(4):
                sl = pl.ds(j * 16, 16)
                sumU[b, sl] = sumU[b, sl] + tmp[b, sl]

        pltpu.async_copy(hops[h].at[irow], tmp, sem).wait()

        @pl.loop(0, BPT)
        def _accI(b):
            for j in range(4):
                sl = pl.ds(j * 16, 16)
                sumI[b, sl] = sumI[b, sl] + tmp[b, sl]

    lane = lax.iota(jnp.int32, 16)

    @pl.loop(0, BPT // 16)
    def _dot(g):
        gsl = pl.ds(g * 16, 16)
        uvec = ubuf[gsl]
        ivec = ibuf[gsl]
        res = jnp.zeros((16,), jnp.float32)
        for bb in range(16):
            b = g * 16 + bb
            accv = jnp.zeros((16,), jnp.float32)
            for j in range(4):
                sl = pl.ds(j * 16, 16)
                accv = accv + sumU[b, sl] * sumI[b, sl]
            s = jnp.sum(accv) * jnp.float32(1.0 / 16.0)
            s = s + ubtab[pl.ds(uvec[bb], 16)][0] + ibtab[pl.ds(ivec[bb], 16)][0]
            res = jnp.where(lane == bb, jnp.full((16,), s, jnp.float32), res)
        outb[gsl] = res

    pltpu.sync_copy(outb, gamma_hbm.at[pl.ds(bbase, BPT)])


_readout = pl.kernel(
    _readout_body,
    out_type=jax.ShapeDtypeStruct((B,), jnp.float32),
    mesh=_mesh,
    compiler_params=pltpu.CompilerParams(use_tc_tiling_on_sc=False,
                                        needs_layout_passes=False),
    scratch_types=[
        pltpu.VMEM((BPT,), jnp.int32),
        pltpu.VMEM((BPT,), jnp.int32),
        pltpu.VMEM((BPT,), jnp.int32),
        pltpu.VMEM((NU + 16,), jnp.float32),
        pltpu.VMEM((NU + 16,), jnp.float32),
        pltpu.VMEM((BPT, D), jnp.float32),
        pltpu.VMEM((BPT, D), jnp.float32),
        pltpu.VMEM((BPT, D), jnp.float32),
        pltpu.VMEM((BPT,), jnp.float32),
        pltpu.SemaphoreType.DMA,
    ],
)


@jax.jit
def kernel(users, items, edge_index, graph_values,
           user_emb, item_emb, user_bias, item_bias):
    src = edge_index[0].astype(jnp.int32)
    dst = edge_index[1].astype(jnp.int32)
    pad = EPAD - E
    src_p = jnp.concatenate([src, jnp.zeros((pad,), jnp.int32)])
    dst_p = jnp.concatenate([dst, jnp.full((pad,), 2 * NU, jnp.int32)])
    w_p = jnp.concatenate([graph_values.astype(jnp.float32),
                           jnp.zeros((pad,), jnp.float32)])

    e0 = jnp.zeros((NN, D), jnp.float32)
    e0 = e0.at[:NU].set(user_emb).at[HP:HP + NU].set(item_emb)

    e1 = _layer(src_p, dst_p, w_p, e0)
    e2 = _layer(src_p, dst_p, w_p, e1)
    e3 = _layer(src_p, dst_p, w_p, e2)

    gamma = _readout(users.astype(jnp.int32), items.astype(jnp.int32),
                     user_bias[:, 0].astype(jnp.float32),
                     item_bias[:, 0].astype(jnp.float32),
                     e0, e1, e2, e3)
    return gamma


# speedup vs baseline: 3.3952x; 1.1789x over previous
import jax
import jax.numpy as jnp
from jax import lax
from jax.experimental import pallas as pl
from jax.experimental.pallas import tpu as pltpu, tpu_sc as plsc

NU = 25000
HP = 25600
NN = 2 * HP
D = 64
E = 800000
NC, NS = 2, 16
CH = 1024
NCHUNK = 49
EPT = NCHUNK * CH
EPAD = NS * EPT
DUMP = NU
RPT = HP // NS
B = 4096
BPT = B // (NC * NS)

_mesh = plsc.VectorSubcoreMesh(core_axis_name="c", subcore_axis_name="s",
                               num_cores=NC, num_subcores=NS)


def _layer_body(src_hbm, dst_hbm, w_hbm, emb_hbm, out_hbm,
                srcb, dstb, wb, rows, gsem, ssem, acc):
    c = lax.axis_index("c")
    s = lax.axis_index("s")

    @pl.loop(0, 64)
    def _zero_rows(k):
        for j in range(4):
            rows[0, k, pl.ds(j * 16, 16)] = jnp.zeros((16,), jnp.float32)

    abase = s * RPT

    @pl.loop(0, RPT // 64)
    def _zero_acc(i):
        pltpu.sync_copy(rows.at[0, pl.ds(0, 64)],
                        acc.at[pl.ds(abase + i * 64, 64)])

    plsc.subcore_barrier()

    dst_lo = c * NU

    @pl.loop(0, NCHUNK)
    def _chunk(ci):
        ebase = s * EPT + ci * CH
        idx_cps = [pltpu.async_copy(src_hbm.at[pl.ds(ebase + r * 128, 128)],
                                    srcb.at[r], gsem) for r in range(8)]
        idx_cps += [pltpu.async_copy(dst_hbm.at[pl.ds(ebase + r * 128, 128)],
                                     dstb.at[r], gsem) for r in range(8)]
        idx_cps.append(pltpu.async_copy(w_hbm.at[pl.ds(ebase, CH)],
                                        wb.at[pl.ds(0, CH)], gsem))
        for cp in idx_cps:
            cp.wait()

        for r in range(8):
            @pl.loop(0, 8)
            def _fix(g, r=r):
                sl = pl.ds(g * 16, 16)
                sv = srcb[r, sl]
                srcb[r, sl] = jnp.where(sv < NU, sv, sv + (HP - NU))
                dv = dstb[r, sl] - dst_lo
                ok = (dv >= 0) & (dv < NU)
                dstb[r, sl] = jnp.where(ok, dv, DUMP)

        def _gather(r):
            return pltpu.async_copy(emb_hbm.at[srcb.at[r]],
                                    rows.at[r & 1], gsem)

        gets = [None] * 8
        puts = [None] * 8
        gets[0] = _gather(0)
        for r in range(8):
            b = r & 1
            gets[r].wait()
            if r >= 1:
                puts[r - 1].wait()
            if r < 7:
                gets[r + 1] = _gather(r + 1)

            @pl.loop(0, 8)
            def _scale(sg, r=r, b=b):
                wvec = wb[pl.ds(r * 128 + sg * 16, 16)]
                for bb in range(16):
                    wsp = jnp.full((16,), wvec[bb], jnp.float32)
                    k = sg * 16 + bb
                    for j in range(4):
                        sl = pl.ds(j * 16, 16)
                        rows[b, k, sl] = rows[b, k, sl] * wsp

            puts[r] = pltpu.async_copy(rows.at[b], acc.at[dstb.at[r]],
                                       ssem, add=True)
        puts[7].wait()

    plsc.subcore_barrier()

    pltpu.sync_copy(acc.at[pl.ds(abase, RPT)],
                    out_hbm.at[pl.ds(c * HP + abase, RPT)])


_layer = pl.kernel(
    _layer_body,
    out_type=jax.ShapeDtypeStruct((NN, D), jnp.float32),
    mesh=_mesh,
    compiler_params=pltpu.CompilerParams(use_tc_tiling_on_sc=False,
                                        needs_layout_passes=False),
    scratch_types=[
        pltpu.VMEM((8, 128), jnp.int32),
        pltpu.VMEM((8, 128), jnp.int32),
        pltpu.VMEM((CH + 16,), jnp.float32),
        pltpu.VMEM((2, 128, D), jnp.float32),
        pltpu.SemaphoreType.DMA,
        pltpu.SemaphoreType.DMA,
        pltpu.VMEM_SHARED((HP, D), jnp.float32),
    ],
)


def _readout_body(users_hbm, items_hbm, ub_hbm, ib_hbm,
                  e0, e1, e2, e3, gamma_hbm,
                  ubuf, ibuf, irow, ubtab, ibtab,
                  sumU, sumI, tmp, outb, sem):
    c = lax.axis_index("c")
    s = lax.axis_index("s")
    wid = s * NC + c
    bbase = wid * BPT

    pltpu.sync_copy(users_hbm.at[pl.ds(bbase, BPT)], ubuf)
    pltpu.sync_copy(items_hbm.at[pl.ds(bbase, BPT)], ibuf)
    pltpu.sync_copy(ub_hbm, ubtab.at[pl.ds(0, NU)])
    pltpu.sync_copy(ib_hbm, ibtab.at[pl.ds(0, NU)])

    @pl.loop(0, BPT // 16)
    def _mkrow(g):
        sl = pl.ds(g * 16, 16)
        irow[sl] = ibuf[sl] + HP

    hops = [e0, e1, e2, e3]
    pltpu.async_copy(hops[0].at[ubuf], sumU, sem).wait()
    pltpu.async_copy(hops[0].at[irow], sumI, sem).wait()
    for h in range(1, 4):
        pltpu.async_copy(hops[h].at[ubuf], tmp, sem).wait()

        @pl.loop(0, BPT)
        def _accU(b):
            for j in range(4):
                sl = pl.ds(j * 16, 16)
                sumU[b, sl] = sumU[b, sl] + tmp[b, sl]

        pltpu.async_copy(hops[h].at[irow], tmp, sem).wait()

        @pl.loop(0, BPT)
        def _accI(b):
            for j in range(4):
                sl = pl.ds(j * 16, 16)
                sumI[b, sl] = sumI[b, sl] + tmp[b, sl]

    lane = lax.iota(jnp.int32, 16)

    @pl.loop(0, BPT // 16)
    def _dot(g):
        gsl = pl.ds(g * 16, 16)
        uvec = ubuf[gsl]
        ivec = ibuf[gsl]
        res = jnp.zeros((16,), jnp.float32)
        for bb in range(16):
            b = g * 16 + bb
            accv = jnp.zeros((16,), jnp.float32)
            for j in range(4):
                sl = pl.ds(j * 16, 16)
                accv = accv + sumU[b, sl] * sumI[b, sl]
            s = jnp.sum(accv) * jnp.float32(1.0 / 16.0)
            s = s + ubtab[pl.ds(uvec[bb], 16)][0] + ibtab[pl.ds(ivec[bb], 16)][0]
            res = jnp.where(lane == bb, jnp.full((16,), s, jnp.float32), res)
        outb[gsl] = res

    pltpu.sync_copy(outb, gamma_hbm.at[pl.ds(bbase, BPT)])


_readout = pl.kernel(
    _readout_body,
    out_type=jax.ShapeDtypeStruct((B,), jnp.float32),
    mesh=_mesh,
    compiler_params=pltpu.CompilerParams(use_tc_tiling_on_sc=False,
                                        needs_layout_passes=False),
    scratch_types=[
        pltpu.VMEM((BPT,), jnp.int32),
        pltpu.VMEM((BPT,), jnp.int32),
        pltpu.VMEM((BPT,), jnp.int32),
        pltpu.VMEM((NU + 16,), jnp.float32),
        pltpu.VMEM((NU + 16,), jnp.float32),
        pltpu.VMEM((BPT, D), jnp.float32),
        pltpu.VMEM((BPT, D), jnp.float32),
        pltpu.VMEM((BPT, D), jnp.float32),
        pltpu.VMEM((BPT,), jnp.float32),
        pltpu.SemaphoreType.DMA,
    ],
)


@jax.jit
def kernel(users, items, edge_index, graph_values,
           user_emb, item_emb, user_bias, item_bias):
    src = edge_index[0].astype(jnp.int32)
    dst = edge_index[1].astype(jnp.int32)
    pad = EPAD - E
    src_p = jnp.concatenate([src, jnp.zeros((pad,), jnp.int32)])
    dst_p = jnp.concatenate([dst, jnp.full((pad,), 2 * NU, jnp.int32)])
    w_p = jnp.concatenate([graph_values.astype(jnp.float32),
                           jnp.zeros((pad,), jnp.float32)])

    e0 = jnp.zeros((NN, D), jnp.float32)
    e0 = e0.at[:NU].set(user_emb).at[HP:HP + NU].set(item_emb)

    e1 = _layer(src_p, dst_p, w_p, e0)
    e2 = _layer(src_p, dst_p, w_p, e1)
    e3 = _layer(src_p, dst_p, w_p, e2)

    gamma = _readout(users.astype(jnp.int32), items.astype(jnp.int32),
                     user_bias[:, 0].astype(jnp.float32),
                     item_bias[:, 0].astype(jnp.float32),
                     e0, e1, e2, e3)
    return gamma


# speedup vs baseline: 3.4100x; 1.0043x over previous
import jax
import jax.numpy as jnp
from jax import lax
from jax.experimental import pallas as pl
from jax.experimental.pallas import tpu as pltpu, tpu_sc as plsc

NU = 25000
HP = 25600
NN = 2 * HP
D = 64
E = 800000
NC, NS = 2, 16
CH = 1024
NCHUNK = 49
EPT = NCHUNK * CH
EPAD = NS * EPT
DUMP = NU
RPT = HP // NS
B = 4096
BPT = B // (NC * NS)

_mesh = plsc.VectorSubcoreMesh(core_axis_name="c", subcore_axis_name="s",
                               num_cores=NC, num_subcores=NS)


def _layer_body(src_hbm, dst_hbm, w_hbm, emb_hbm, out_hbm,
                srcb, dstb, wb, rows, gsem, ssem, acc):
    c = lax.axis_index("c")
    s = lax.axis_index("s")

    @pl.loop(0, 64)
    def _zero_rows(k):
        for j in range(4):
            rows[0, k, pl.ds(j * 16, 16)] = jnp.zeros((16,), jnp.float32)

    abase = s * RPT

    @pl.loop(0, RPT // 64)
    def _zero_acc(i):
        pltpu.sync_copy(rows.at[0, pl.ds(0, 64)],
                        acc.at[pl.ds(abase + i * 64, 64)])

    plsc.subcore_barrier()

    dst_lo = c * NU

    @pl.loop(0, NCHUNK)
    def _chunk(ci):
        ebase = s * EPT + ci * CH
        idx_cps = [pltpu.async_copy(src_hbm.at[pl.ds(ebase + r * 128, 128)],
                                    srcb.at[r], gsem) for r in range(8)]
        idx_cps += [pltpu.async_copy(dst_hbm.at[pl.ds(ebase + r * 128, 128)],
                                     dstb.at[r], gsem) for r in range(8)]
        idx_cps.append(pltpu.async_copy(w_hbm.at[pl.ds(ebase, CH)],
                                        wb.at[pl.ds(0, CH)], gsem))
        for cp in idx_cps:
            cp.wait()

        for r in range(8):
            @pl.loop(0, 8)
            def _fix(g, r=r):
                sl = pl.ds(g * 16, 16)
                sv = srcb[r, sl]
                srcb[r, sl] = jnp.where(sv < NU, sv, sv + (HP - NU))
                dv = dstb[r, sl] - dst_lo
                ok = (dv >= 0) & (dv < NU)
                dstb[r, sl] = jnp.where(ok, dv, DUMP)

        def _gather(r):
            return pltpu.async_copy(emb_hbm.at[srcb.at[r]],
                                    rows.at[r & 1], gsem)

        gets = [None] * 8
        puts = [None] * 8
        gets[0] = _gather(0)
        for r in range(8):
            b = r & 1
            gets[r].wait()
            if r >= 1:
                puts[r - 1].wait()
            if r < 7:
                gets[r + 1] = _gather(r + 1)

            @pl.loop(0, 8)
            def _scale(sg, r=r, b=b):
                wvec = wb[pl.ds(r * 128 + sg * 16, 16)]
                dn = lax.GatherDimensionNumbers(offset_dims=(),
                                                collapsed_slice_dims=(0,),
                                                start_index_map=(0,))
                for bb in range(16):
                    idx = jnp.full((16, 1), bb, jnp.int32)
                    wsp = lax.gather(wvec, idx, dn, (1,),
                                     mode=lax.GatherScatterMode.PROMISE_IN_BOUNDS)
                    k = sg * 16 + bb
                    for j in range(4):
                        sl = pl.ds(j * 16, 16)
                        rows[b, k, sl] = rows[b, k, sl] * wsp

            puts[r] = pltpu.async_copy(rows.at[b], acc.at[dstb.at[r]],
                                       ssem, add=True)
        puts[7].wait()

    plsc.subcore_barrier()

    pltpu.sync_copy(acc.at[pl.ds(abase, RPT)],
                    out_hbm.at[pl.ds(c * HP + abase, RPT)])


_layer = pl.kernel(
    _layer_body,
    out_type=jax.ShapeDtypeStruct((NN, D), jnp.float32),
    mesh=_mesh,
    compiler_params=pltpu.CompilerParams(use_tc_tiling_on_sc=False,
                                        needs_layout_passes=False),
    scratch_types=[
        pltpu.VMEM((8, 128), jnp.int32),
        pltpu.VMEM((8, 128), jnp.int32),
        pltpu.VMEM((CH + 16,), jnp.float32),
        pltpu.VMEM((2, 128, D), jnp.float32),
        pltpu.SemaphoreType.DMA,
        pltpu.SemaphoreType.DMA,
        pltpu.VMEM_SHARED((HP, D), jnp.float32),
    ],
)


def _readout_body(users_hbm, items_hbm, ub_hbm, ib_hbm,
                  e0, e1, e2, e3, gamma_hbm,
                  ubuf, ibuf, irow, ubtab, ibtab,
                  sumU, sumI, tmp, outb, sem):
    c = lax.axis_index("c")
    s = lax.axis_index("s")
    wid = s * NC + c
    bbase = wid * BPT

    pltpu.sync_copy(users_hbm.at[pl.ds(bbase, BPT)], ubuf)
    pltpu.sync_copy(items_hbm.at[pl.ds(bbase, BPT)], ibuf)
    pltpu.sync_copy(ub_hbm, ubtab.at[pl.ds(0, NU)])
    pltpu.sync_copy(ib_hbm, ibtab.at[pl.ds(0, NU)])

    @pl.loop(0, BPT // 16)
    def _mkrow(g):
        sl = pl.ds(g * 16, 16)
        irow[sl] = ibuf[sl] + HP

    hops = [e0, e1, e2, e3]
    pltpu.async_copy(hops[0].at[ubuf], sumU, sem).wait()
    pltpu.async_copy(hops[0].at[irow], sumI, sem).wait()
    for h in range(1, 4):
        pltpu.async_copy(hops[h].at[ubuf], tmp, sem).wait()

        @pl.loop(0, BPT)
        def _accU(b):
            for j in range(4):
                sl = pl.ds(j * 16, 16)
                sumU[b, sl] = sumU[b, sl] + tmp[b, sl]

        pltpu.async_copy(hops[h].at[irow], tmp, sem).wait()

        @pl.loop(0, BPT)
        def _accI(b):
            for j in range(4):
                sl = pl.ds(j * 16, 16)
                sumI[b, sl] = sumI[b, sl] + tmp[b, sl]

    lane = lax.iota(jnp.int32, 16)

    @pl.loop(0, BPT // 16)
    def _dot(g):
        gsl = pl.ds(g * 16, 16)
        uvec = ubuf[gsl]
        ivec = ibuf[gsl]
        res = jnp.zeros((16,), jnp.float32)
        for bb in range(16):
            b = g * 16 + bb
            accv = jnp.zeros((16,), jnp.float32)
            for j in range(4):
                sl = pl.ds(j * 16, 16)
                accv = accv + sumU[b, sl] * sumI[b, sl]
            s = jnp.sum(accv) * jnp.float32(1.0 / 16.0)
            s = s + ubtab[pl.ds(uvec[bb], 16)][0] + ibtab[pl.ds(ivec[bb], 16)][0]
            res = jnp.where(lane == bb, jnp.full((16,), s, jnp.float32), res)
        outb[gsl] = res

    pltpu.sync_copy(outb, gamma_hbm.at[pl.ds(bbase, BPT)])


_readout = pl.kernel(
    _readout_body,
    out_type=jax.ShapeDtypeStruct((B,), jnp.float32),
    mesh=_mesh,
    compiler_params=pltpu.CompilerParams(use_tc_tiling_on_sc=False,
                                        needs_layout_passes=False),
    scratch_types=[
        pltpu.VMEM((BPT,), jnp.int32),
        pltpu.VMEM((BPT,), jnp.int32),
        pltpu.VMEM((BPT,), jnp.int32),
        pltpu.VMEM((NU + 16,), jnp.float32),
        pltpu.VMEM((NU + 16,), jnp.float32),
        pltpu.VMEM((BPT, D), jnp.float32),
        pltpu.VMEM((BPT, D), jnp.float32),
        pltpu.VMEM((BPT, D), jnp.float32),
        pltpu.VMEM((BPT,), jnp.float32),
        pltpu.SemaphoreType.DMA,
    ],
)


@jax.jit
def kernel(users, items, edge_index, graph_values,
           user_emb, item_emb, user_bias, item_bias):
    src = edge_index[0].astype(jnp.int32)
    dst = edge_index[1].astype(jnp.int32)
    pad = EPAD - E
    src_p = jnp.concatenate([src, jnp.zeros((pad,), jnp.int32)])
    dst_p = jnp.concatenate([dst, jnp.full((pad,), 2 * NU, jnp.int32)])
    w_p = jnp.concatenate([graph_values.astype(jnp.float32),
                           jnp.zeros((pad,), jnp.float32)])

    e0 = jnp.zeros((NN, D), jnp.float32)
    e0 = e0.at[:NU].set(user_emb).at[HP:HP + NU].set(item_emb)

    e1 = _layer(src_p, dst_p, w_p, e0)
    e2 = _layer(src_p, dst_p, w_p, e1)
    e3 = _layer(src_p, dst_p, w_p, e2)

    gamma = _readout(users.astype(jnp.int32), items.astype(jnp.int32),
                     user_bias[:, 0].astype(jnp.float32),
                     item_bias[:, 0].astype(jnp.float32),
                     e0, e1, e2, e3)
    return gamma


# speedup vs baseline: 5.6620x; 1.6604x over previous
import jax
import jax.numpy as jnp
from jax import lax
from jax.experimental import pallas as pl
from jax.experimental.pallas import tpu as pltpu, tpu_sc as plsc

NU = 25000
HP = 25600
NN = 2 * HP
D = 64
E = 800000
NC, NS = 2, 16
CH = 1024
NCHUNK = 49
EPT = NCHUNK * CH
EPAD = NS * EPT
DUMP = NU
RPT = HP // NS
B = 4096
BPT = B // (NC * NS)

_mesh = plsc.VectorSubcoreMesh(core_axis_name="c", subcore_axis_name="s",
                               num_cores=NC, num_subcores=NS)


def _layer_body(src_hbm, dst_hbm, w_hbm, emb_hbm, out_hbm,
                srcb, dstb, wb, rows, gsem, ssem, acc):
    c = lax.axis_index("c")
    s = lax.axis_index("s")

    @pl.loop(0, 64)
    def _zero_rows(k):
        for j in range(4):
            rows[0, k, pl.ds(j * 16, 16)] = jnp.zeros((16,), jnp.float32)

    abase = s * RPT

    @pl.loop(0, RPT // 64)
    def _zero_acc(i):
        pltpu.sync_copy(rows.at[0, pl.ds(0, 64)],
                        acc.at[pl.ds(abase + i * 64, 64)])

    plsc.subcore_barrier()

    dst_lo = c * NU

    @pl.loop(0, NCHUNK)
    def _chunk(ci):
        ebase = s * EPT + ci * CH
        idx_cps = [pltpu.async_copy(src_hbm.at[pl.ds(ebase + r * 128, 128)],
                                    srcb.at[r], gsem) for r in range(8)]
        idx_cps += [pltpu.async_copy(dst_hbm.at[pl.ds(ebase + r * 128, 128)],
                                     dstb.at[r], gsem) for r in range(8)]
        idx_cps.append(pltpu.async_copy(w_hbm.at[pl.ds(ebase, CH)],
                                        wb.at[pl.ds(0, CH)], gsem))
        for cp in idx_cps:
            cp.wait()

        for r in range(8):
            @pl.loop(0, 8)
            def _fix(g, r=r):
                sl = pl.ds(g * 16, 16)
                sv = srcb[r, sl]
                srcb[r, sl] = jnp.where(sv < NU, sv, sv + (HP - NU))
                dv = dstb[r, sl] - dst_lo
                ok = (dv >= 0) & (dv < NU)
                dstb[r, sl] = jnp.where(ok, dv, DUMP)

        def _gather(r):
            return pltpu.async_copy(emb_hbm.at[srcb.at[r]],
                                    rows.at[r & 1], gsem)

        gets = [None] * 8
        puts = [None] * 8
        gets[0] = _gather(0)
        for r in range(8):
            b = r & 1
            gets[r].wait()
            if r >= 1:
                puts[r - 1].wait()
            if r < 7:
                gets[r + 1] = _gather(r + 1)

            @plsc.parallel_loop(0, 8, unroll=2)
            def _scale(sg, r=r, b=b):
                wvec = wb[pl.ds(r * 128 + sg * 16, 16)]
                dn = lax.GatherDimensionNumbers(offset_dims=(),
                                                collapsed_slice_dims=(0,),
                                                start_index_map=(0,))
                for bb in range(16):
                    idx = jnp.full((16, 1), bb, jnp.int32)
                    wsp = lax.gather(wvec, idx, dn, (1,),
                                     mode=lax.GatherScatterMode.PROMISE_IN_BOUNDS)
                    k = sg * 16 + bb
                    for j in range(4):
                        sl = pl.ds(j * 16, 16)
                        rows[b, k, sl] = rows[b, k, sl] * wsp

            puts[r] = pltpu.async_copy(rows.at[b], acc.at[dstb.at[r]],
                                       ssem, add=True)
        puts[7].wait()

    plsc.subcore_barrier()

    pltpu.sync_copy(acc.at[pl.ds(abase, RPT)],
                    out_hbm.at[pl.ds(c * HP + abase, RPT)])


_layer = pl.kernel(
    _layer_body,
    out_type=jax.ShapeDtypeStruct((NN, D), jnp.float32),
    mesh=_mesh,
    compiler_params=pltpu.CompilerParams(use_tc_tiling_on_sc=False,
                                        needs_layout_passes=False),
    scratch_types=[
        pltpu.VMEM((8, 128), jnp.int32),
        pltpu.VMEM((8, 128), jnp.int32),
        pltpu.VMEM((CH + 16,), jnp.float32),
        pltpu.VMEM((2, 128, D), jnp.float32),
        pltpu.SemaphoreType.DMA,
        pltpu.SemaphoreType.DMA,
        pltpu.VMEM_SHARED((HP, D), jnp.float32),
    ],
)


def _readout_body(users_hbm, items_hbm, ub_hbm, ib_hbm,
                  e0, e1, e2, e3, gamma_hbm,
                  ubuf, ibuf, irow, ubtab, ibtab,
                  sumU, sumI, tmp, outb, sem):
    c = lax.axis_index("c")
    s = lax.axis_index("s")
    wid = s * NC + c
    bbase = wid * BPT

    pltpu.sync_copy(users_hbm.at[pl.ds(bbase, BPT)], ubuf)
    pltpu.sync_copy(items_hbm.at[pl.ds(bbase, BPT)], ibuf)
    pltpu.sync_copy(ub_hbm, ubtab.at[pl.ds(0, NU)])
    pltpu.sync_copy(ib_hbm, ibtab.at[pl.ds(0, NU)])

    @pl.loop(0, BPT // 16)
    def _mkrow(g):
        sl = pl.ds(g * 16, 16)
        irow[sl] = ibuf[sl] + HP

    hops = [e0, e1, e2, e3]
    pltpu.async_copy(hops[0].at[ubuf], sumU, sem).wait()
    pltpu.async_copy(hops[0].at[irow], sumI, sem).wait()
    for h in range(1, 4):
        pltpu.async_copy(hops[h].at[ubuf], tmp, sem).wait()

        @pl.loop(0, BPT)
        def _accU(b):
            for j in range(4):
                sl = pl.ds(j * 16, 16)
                sumU[b, sl] = sumU[b, sl] + tmp[b, sl]

        pltpu.async_copy(hops[h].at[irow], tmp, sem).wait()

        @pl.loop(0, BPT)
        def _accI(b):
            for j in range(4):
                sl = pl.ds(j * 16, 16)
                sumI[b, sl] = sumI[b, sl] + tmp[b, sl]

    lane = lax.iota(jnp.int32, 16)

    @pl.loop(0, BPT // 16)
    def _dot(g):
        gsl = pl.ds(g * 16, 16)
        uvec = ubuf[gsl]
        ivec = ibuf[gsl]
        res = jnp.zeros((16,), jnp.float32)
        for bb in range(16):
            b = g * 16 + bb
            accv = jnp.zeros((16,), jnp.float32)
            for j in range(4):
                sl = pl.ds(j * 16, 16)
                accv = accv + sumU[b, sl] * sumI[b, sl]
            s = jnp.sum(accv) * jnp.float32(1.0 / 16.0)
            s = s + ubtab[pl.ds(uvec[bb], 16)][0] + ibtab[pl.ds(ivec[bb], 16)][0]
            res = jnp.where(lane == bb, jnp.full((16,), s, jnp.float32), res)
        outb[gsl] = res

    pltpu.sync_copy(outb, gamma_hbm.at[pl.ds(bbase, BPT)])


_readout = pl.kernel(
    _readout_body,
    out_type=jax.ShapeDtypeStruct((B,), jnp.float32),
    mesh=_mesh,
    compiler_params=pltpu.CompilerParams(use_tc_tiling_on_sc=False,
                                        needs_layout_passes=False),
    scratch_types=[
        pltpu.VMEM((BPT,), jnp.int32),
        pltpu.VMEM((BPT,), jnp.int32),
        pltpu.VMEM((BPT,), jnp.int32),
        pltpu.VMEM((NU + 16,), jnp.float32),
        pltpu.VMEM((NU + 16,), jnp.float32),
        pltpu.VMEM((BPT, D), jnp.float32),
        pltpu.VMEM((BPT, D), jnp.float32),
        pltpu.VMEM((BPT, D), jnp.float32),
        pltpu.VMEM((BPT,), jnp.float32),
        pltpu.SemaphoreType.DMA,
    ],
)


@jax.jit
def kernel(users, items, edge_index, graph_values,
           user_emb, item_emb, user_bias, item_bias):
    src = edge_index[0].astype(jnp.int32)
    dst = edge_index[1].astype(jnp.int32)
    pad = EPAD - E
    src_p = jnp.concatenate([src, jnp.zeros((pad,), jnp.int32)])
    dst_p = jnp.concatenate([dst, jnp.full((pad,), 2 * NU, jnp.int32)])
    w_p = jnp.concatenate([graph_values.astype(jnp.float32),
                           jnp.zeros((pad,), jnp.float32)])

    e0 = jnp.zeros((NN, D), jnp.float32)
    e0 = e0.at[:NU].set(user_emb).at[HP:HP + NU].set(item_emb)

    e1 = _layer(src_p, dst_p, w_p, e0)
    e2 = _layer(src_p, dst_p, w_p, e1)
    e3 = _layer(src_p, dst_p, w_p, e2)

    gamma = _readout(users.astype(jnp.int32), items.astype(jnp.int32),
                     user_bias[:, 0].astype(jnp.float32),
                     item_bias[:, 0].astype(jnp.float32),
                     e0, e1, e2, e3)
    return gamma


# speedup vs baseline: 5.7044x; 1.0075x over previous
import jax
import jax.numpy as jnp
from jax import lax
from jax.experimental import pallas as pl
from jax.experimental.pallas import tpu as pltpu, tpu_sc as plsc

NU = 25000
HP = 25600
NN = 2 * HP
D = 64
E = 800000
NC, NS = 2, 16
CH = 1024
NCHUNK = 49
EPT = NCHUNK * CH
EPAD = NS * EPT
DUMP = NU
RPT = HP // NS
B = 4096
BPT = B // (NC * NS)

_mesh = plsc.VectorSubcoreMesh(core_axis_name="c", subcore_axis_name="s",
                               num_cores=NC, num_subcores=NS)


def _layer_body(src_hbm, dst_hbm, w_hbm, emb_hbm, out_hbm,
                srcb, dstb, wb, rows, gsem, ssem, acc):
    c = lax.axis_index("c")
    s = lax.axis_index("s")

    @pl.loop(0, 64)
    def _zero_rows(k):
        for j in range(4):
            rows[0, k, pl.ds(j * 16, 16)] = jnp.zeros((16,), jnp.float32)

    abase = s * RPT

    @pl.loop(0, RPT // 64)
    def _zero_acc(i):
        pltpu.sync_copy(rows.at[0, pl.ds(0, 64)],
                        acc.at[pl.ds(abase + i * 64, 64)])

    plsc.subcore_barrier()

    dst_lo = c * NU

    @pl.loop(0, NCHUNK)
    def _chunk(ci):
        ebase = s * EPT + ci * CH
        idx_cps = [pltpu.async_copy(src_hbm.at[pl.ds(ebase + r * 128, 128)],
                                    srcb.at[r], gsem) for r in range(8)]
        idx_cps += [pltpu.async_copy(dst_hbm.at[pl.ds(ebase + r * 128, 128)],
                                     dstb.at[r], gsem) for r in range(8)]
        idx_cps.append(pltpu.async_copy(w_hbm.at[pl.ds(ebase, CH)],
                                        wb.at[pl.ds(0, CH)], gsem))
        for cp in idx_cps:
            cp.wait()

        for r in range(8):
            @pl.loop(0, 8)
            def _fix(g, r=r):
                sl = pl.ds(g * 16, 16)
                sv = srcb[r, sl]
                srcb[r, sl] = jnp.where(sv < NU, sv, sv + (HP - NU))
                dv = dstb[r, sl] - dst_lo
                ok = (dv >= 0) & (dv < NU)
                dstb[r, sl] = jnp.where(ok, dv, DUMP)

        def _gather(r):
            return pltpu.async_copy(emb_hbm.at[srcb.at[r]],
                                    rows.at[r % 3], gsem)

        gets = [None] * 8
        puts = [None] * 8
        gets[0] = _gather(0)
        gets[1] = _gather(1)
        for r in range(8):
            b = r % 3
            gets[r].wait()
            if r >= 1:
                puts[r - 1].wait()
            if r + 2 < 8:
                gets[r + 2] = _gather(r + 2)

            @plsc.parallel_loop(0, 8, unroll=2)
            def _scale(sg, r=r, b=b):
                wvec = wb[pl.ds(r * 128 + sg * 16, 16)]
                dn = lax.GatherDimensionNumbers(offset_dims=(),
                                                collapsed_slice_dims=(0,),
                                                start_index_map=(0,))
                for bb in range(16):
                    idx = jnp.full((16, 1), bb, jnp.int32)
                    wsp = lax.gather(wvec, idx, dn, (1,),
                                     mode=lax.GatherScatterMode.PROMISE_IN_BOUNDS)
                    k = sg * 16 + bb
                    for j in range(4):
                        sl = pl.ds(j * 16, 16)
                        rows[b, k, sl] = rows[b, k, sl] * wsp

            puts[r] = pltpu.async_copy(rows.at[b], acc.at[dstb.at[r]],
                                       ssem, add=True)
        puts[7].wait()

    plsc.subcore_barrier()

    pltpu.sync_copy(acc.at[pl.ds(abase, RPT)],
                    out_hbm.at[pl.ds(c * HP + abase, RPT)])


_layer = pl.kernel(
    _layer_body,
    out_type=jax.ShapeDtypeStruct((NN, D), jnp.float32),
    mesh=_mesh,
    compiler_params=pltpu.CompilerParams(use_tc_tiling_on_sc=False,
                                        needs_layout_passes=False),
    scratch_types=[
        pltpu.VMEM((8, 128), jnp.int32),
        pltpu.VMEM((8, 128), jnp.int32),
        pltpu.VMEM((CH + 16,), jnp.float32),
        pltpu.VMEM((3, 128, D), jnp.float32),
        pltpu.SemaphoreType.DMA,
        pltpu.SemaphoreType.DMA,
        pltpu.VMEM_SHARED((HP, D), jnp.float32),
    ],
)


def _readout_body(users_hbm, items_hbm, ub_hbm, ib_hbm,
                  e0, e1, e2, e3, gamma_hbm,
                  ubuf, ibuf, irow, ubtab, ibtab,
                  sumU, sumI, tmp, outb, sem):
    c = lax.axis_index("c")
    s = lax.axis_index("s")
    wid = s * NC + c
    bbase = wid * BPT

    pltpu.sync_copy(users_hbm.at[pl.ds(bbase, BPT)], ubuf)
    pltpu.sync_copy(items_hbm.at[pl.ds(bbase, BPT)], ibuf)
    pltpu.sync_copy(ub_hbm, ubtab.at[pl.ds(0, NU)])
    pltpu.sync_copy(ib_hbm, ibtab.at[pl.ds(0, NU)])

    @pl.loop(0, BPT // 16)
    def _mkrow(g):
        sl = pl.ds(g * 16, 16)
        irow[sl] = ibuf[sl] + HP

    hops = [e0, e1, e2, e3]
    pltpu.async_copy(hops[0].at[ubuf], sumU, sem).wait()
    pltpu.async_copy(hops[0].at[irow], sumI, sem).wait()
    for h in range(1, 4):
        pltpu.async_copy(hops[h].at[ubuf], tmp, sem).wait()

        @pl.loop(0, BPT)
        def _accU(b):
            for j in range(4):
                sl = pl.ds(j * 16, 16)
                sumU[b, sl] = sumU[b, sl] + tmp[b, sl]

        pltpu.async_copy(hops[h].at[irow], tmp, sem).wait()

        @pl.loop(0, BPT)
        def _accI(b):
            for j in range(4):
                sl = pl.ds(j * 16, 16)
                sumI[b, sl] = sumI[b, sl] + tmp[b, sl]

    lane = lax.iota(jnp.int32, 16)

    @pl.loop(0, BPT // 16)
    def _dot(g):
        gsl = pl.ds(g * 16, 16)
        uvec = ubuf[gsl]
        ivec = ibuf[gsl]
        res = jnp.zeros((16,), jnp.float32)
        for bb in range(16):
            b = g * 16 + bb
            accv = jnp.zeros((16,), jnp.float32)
            for j in range(4):
                sl = pl.ds(j * 16, 16)
                accv = accv + sumU[b, sl] * sumI[b, sl]
            s = jnp.sum(accv) * jnp.float32(1.0 / 16.0)
            s = s + ubtab[pl.ds(uvec[bb], 16)][0] + ibtab[pl.ds(ivec[bb], 16)][0]
            res = jnp.where(lane == bb, jnp.full((16,), s, jnp.float32), res)
        outb[gsl] = res

    pltpu.sync_copy(outb, gamma_hbm.at[pl.ds(bbase, BPT)])


_readout = pl.kernel(
    _readout_body,
    out_type=jax.ShapeDtypeStruct((B,), jnp.float32),
    mesh=_mesh,
    compiler_params=pltpu.CompilerParams(use_tc_tiling_on_sc=False,
                                        needs_layout_passes=False),
    scratch_types=[
        pltpu.VMEM((BPT,), jnp.int32),
        pltpu.VMEM((BPT,), jnp.int32),
        pltpu.VMEM((BPT,), jnp.int32),
        pltpu.VMEM((NU + 16,), jnp.float32),
        pltpu.VMEM((NU + 16,), jnp.float32),
        pltpu.VMEM((BPT, D), jnp.float32),
        pltpu.VMEM((BPT, D), jnp.float32),
        pltpu.VMEM((BPT, D), jnp.float32),
        pltpu.VMEM((BPT,), jnp.float32),
        pltpu.SemaphoreType.DMA,
    ],
)


@jax.jit
def kernel(users, items, edge_index, graph_values,
           user_emb, item_emb, user_bias, item_bias):
    src = edge_index[0].astype(jnp.int32)
    dst = edge_index[1].astype(jnp.int32)
    pad = EPAD - E
    src_p = jnp.concatenate([src, jnp.zeros((pad,), jnp.int32)])
    dst_p = jnp.concatenate([dst, jnp.full((pad,), 2 * NU, jnp.int32)])
    w_p = jnp.concatenate([graph_values.astype(jnp.float32),
                           jnp.zeros((pad,), jnp.float32)])

    e0 = jnp.zeros((NN, D), jnp.float32)
    e0 = e0.at[:NU].set(user_emb).at[HP:HP + NU].set(item_emb)

    e1 = _layer(src_p, dst_p, w_p, e0)
    e2 = _layer(src_p, dst_p, w_p, e1)
    e3 = _layer(src_p, dst_p, w_p, e2)

    gamma = _readout(users.astype(jnp.int32), items.astype(jnp.int32),
                     user_bias[:, 0].astype(jnp.float32),
                     item_bias[:, 0].astype(jnp.float32),
                     e0, e1, e2, e3)
    return gamma
